# baseline (speedup 1.0000x reference)
import functools

import jax
import jax.numpy as jnp
from jax import lax
from jax.experimental import pallas as pl
from jax.experimental.pallas import tpu as pltpu
from jax.experimental.pallas import tpu_sc as plsc

N = 10000
F = 128
E = 320000
G = 64
NC, NS = 2, 16
NW = NC * NS
EPW = E // NW
K = 128
NCHUNK = 80
NCHUNKP = NCHUNK
NPAD = 10240
TPT = NPAD // NS
R = 2000
NB = N // R

_mesh = plsc.VectorSubcoreMesh(core_axis_name="c", subcore_axis_name="s")


@functools.partial(
    pl.kernel,
    out_type=jax.ShapeDtypeStruct((NC, NPAD), jnp.float32),
    mesh=_mesh,
    scratch_types=[
        pltpu.VMEM((NCHUNKP, K), jnp.int32),
        pltpu.VMEM((K,), jnp.float32),
        pltpu.VMEM_SHARED((NPAD,), jnp.float32),
    ],
)
def _sc_degree(dst_hbm, z_hbm, out_hbm, dst_v, ones_v, acc_sh):
    c = lax.axis_index("c")
    s = lax.axis_index("s")
    w = c * NS + s
    pltpu.sync_copy(z_hbm.at[pl.ds(s * TPT, TPT)], acc_sh.at[pl.ds(s * TPT, TPT)])
    pltpu.sync_copy(dst_hbm.at[w], dst_v)
    for i in range(K // 16):
        ones_v[pl.ds(i * 16, 16)] = jnp.ones((16,), jnp.float32)
    plsc.subcore_barrier()

    def body(j, carry):
        pltpu.sync_copy(ones_v, acc_sh.at[dst_v.at[j]], add=True)
        return carry

    lax.fori_loop(0, NCHUNK, body, 0)
    plsc.subcore_barrier()
    pltpu.sync_copy(acc_sh.at[pl.ds(s * TPT, TPT)],
                    out_hbm.at[c, pl.ds(s * TPT, TPT)])


@functools.partial(
    pl.kernel,
    out_type=jax.ShapeDtypeStruct((NC, NPAD, F), jnp.float32),
    mesh=_mesh,
    scratch_types=[
        pltpu.VMEM((NCHUNKP, K), jnp.int32),
        pltpu.VMEM((NCHUNKP, K), jnp.int32),
        pltpu.VMEM((K, F), jnp.float32),
        pltpu.VMEM_SHARED((NPAD, F), jnp.float32),
        pltpu.SemaphoreType.DMA,
    ],
)
def _sc_edge(hs_hbm, src_hbm, dst_hbm, z_hbm, out_hbm,
             src_v, dst_v, rows_v, acc_sh, sem):
    c = lax.axis_index("c")
    s = lax.axis_index("s")
    w = c * NS + s
    pltpu.sync_copy(z_hbm.at[pl.ds(s * TPT, TPT)], acc_sh.at[pl.ds(s * TPT, TPT)])
    pltpu.sync_copy(src_hbm.at[w], src_v)
    pltpu.sync_copy(dst_hbm.at[w], dst_v)
    plsc.subcore_barrier()

    def body(j, carry):
        pltpu.async_copy(hs_hbm.at[src_v.at[j]], rows_v, sem).wait()
        pltpu.sync_copy(rows_v, acc_sh.at[dst_v.at[j]], add=True)
        return carry

    lax.fori_loop(0, NCHUNK, body, 0)
    plsc.subcore_barrier()
    pltpu.sync_copy(acc_sh.at[pl.ds(s * TPT, TPT)],
                    out_hbm.at[c, pl.ds(s * TPT, TPT)])


def _tc_first(x, W1, cnt_t):
    def body(x_ref, w_ref, c_ref, hs_ref, dis_ref):
        deg = 1.0 + c_ref[:, 0:1] + c_ref[:, 1:2]
        dis = lax.rsqrt(deg)
        h = jnp.dot(x_ref[...], w_ref[...], preferred_element_type=jnp.float32)
        hs_ref[...] = dis * h
        dis_ref[...] = dis

    return pl.pallas_call(
        body,
        grid=(NB,),
        in_specs=[
            pl.BlockSpec((R, F), lambda i: (i, 0)),
            pl.BlockSpec((F, F), lambda i: (0, 0)),
            pl.BlockSpec((R, 2), lambda i: (i, 0)),
        ],
        out_specs=[
            pl.BlockSpec((R, F), lambda i: (i, 0)),
            pl.BlockSpec((R, 1), lambda i: (i, 0)),
        ],
        out_shape=[
            jax.ShapeDtypeStruct((N, F), jnp.float32),
            jax.ShapeDtypeStruct((N, 1), jnp.float32),
        ],
    )(x, W1, cnt_t)


def _tc_mid(p, hs1, dis, b1r, W2):
    def body(p_ref, hs_ref, dis_ref, b_ref, w_ref, out_ref):
        sm = p_ref[0, :, :] + p_ref[1, :, :] + hs_ref[...]
        a = jnp.maximum(dis_ref[...] * sm + b_ref[...], 0.0)
        out_ref[...] = dis_ref[...] * jnp.dot(
            a, w_ref[...], preferred_element_type=jnp.float32)

    return pl.pallas_call(
        body,
        grid=(NB,),
        in_specs=[
            pl.BlockSpec((NC, R, F), lambda i: (0, i, 0)),
            pl.BlockSpec((R, F), lambda i: (i, 0)),
            pl.BlockSpec((R, 1), lambda i: (i, 0)),
            pl.BlockSpec((1, F), lambda i: (0, 0)),
            pl.BlockSpec((F, F), lambda i: (0, 0)),
        ],
        out_specs=pl.BlockSpec((R, F), lambda i: (i, 0)),
        out_shape=jax.ShapeDtypeStruct((N, F), jnp.float32),
    )(p, hs1, dis, b1r, W2)


def _tc_final(q, hs2, dis, b2r, batchf, L1W, l1br, L2W, l2br):
    def body(q_ref, hs_ref, dis_ref, b_ref, bf_ref, w1_ref, bb1_ref,
             w2_ref, bb2_ref, out_ref, G_acc, C_acc):
        i = pl.program_id(0)

        @pl.when(i == 0)
        def _():
            G_acc[...] = jnp.zeros_like(G_acc)
            C_acc[...] = jnp.zeros_like(C_acc)

        sm = q_ref[0, :, :] + q_ref[1, :, :] + hs_ref[...]
        h2 = jnp.maximum(dis_ref[...] * sm + b_ref[...], 0.0)
        gid = lax.broadcasted_iota(jnp.int32, (R, G), 1).astype(jnp.float32)
        oh = (bf_ref[...] == gid).astype(jnp.float32)
        G_acc[...] += lax.dot_general(
            oh, h2, (((0,), (0,)), ((), ())),
            preferred_element_type=jnp.float32)
        C_acc[...] += lax.dot_general(
            oh, jnp.ones((R, 1), jnp.float32), (((0,), (0,)), ((), ())),
            preferred_element_type=jnp.float32)

        @pl.when(i == NB - 1)
        def _():
            g = G_acc[...] / jnp.maximum(C_acc[...], 1.0)
            z = jnp.maximum(
                jnp.dot(g, w1_ref[...], preferred_element_type=jnp.float32)
                + bb1_ref[...], 0.0)
            out_ref[...] = jnp.dot(
                z, w2_ref[...], preferred_element_type=jnp.float32) + bb2_ref[...]

    return pl.pallas_call(
        body,
        grid=(NB,),
        in_specs=[
            pl.BlockSpec((NC, R, F), lambda i: (0, i, 0)),
            pl.BlockSpec((R, F), lambda i: (i, 0)),
            pl.BlockSpec((R, 1), lambda i: (i, 0)),
            pl.BlockSpec((1, F), lambda i: (0, 0)),
            pl.BlockSpec((R, 1), lambda i: (i, 0)),
            pl.BlockSpec((F, 64), lambda i: (0, 0)),
            pl.BlockSpec((1, 64), lambda i: (0, 0)),
            pl.BlockSpec((64, 40), lambda i: (0, 0)),
            pl.BlockSpec((1, 40), lambda i: (0, 0)),
        ],
        out_specs=pl.BlockSpec((G, 40), lambda i: (0, 0)),
        out_shape=jax.ShapeDtypeStruct((G, 40), jnp.float32),
        scratch_shapes=[
            pltpu.VMEM((G, F), jnp.float32),
            pltpu.VMEM((G, 1), jnp.float32),
        ],
    )(q, hs2, dis, b2r, batchf, L1W, l1br, L2W, l2br)


def kernel(x, edge_index, batch, W1, b1, W2, b2, L1W, L1b, L2W, L2b):
    pad = NCHUNKP * K - EPW
    src = jnp.pad(edge_index[0].reshape(NW, EPW),
                  ((0, 0), (0, pad))).reshape(NW, NCHUNKP, K)
    dst = jnp.pad(edge_index[1].reshape(NW, EPW), ((0, 0), (0, pad)),
                  constant_values=NPAD - 1).reshape(NW, NCHUNKP, K)
    z1 = jnp.zeros((NPAD,), jnp.float32)
    z2 = jnp.zeros((NPAD, F), jnp.float32)
    b1r = b1.reshape(1, F)
    b2r = b2.reshape(1, F)
    l1br = L1b.reshape(1, 64)
    l2br = L2b.reshape(1, 40)
    batchf = batch.astype(jnp.float32).reshape(N, 1)

    cnt = _sc_degree(dst, z1)
    cnt_t = cnt.T
    hs1, dis = _tc_first(x, W1, cnt_t)
    p = _sc_edge(hs1, src, dst, z2)
    hs2 = _tc_mid(p, hs1, dis, b1r, W2)
    q = _sc_edge(hs2, src, dst, z2)
    return _tc_final(q, hs2, dis, b2r, batchf, L1W, l1br, L2W, l2br)


# speedup vs baseline: 1.6866x; 1.6866x over previous
import functools

import jax
import jax.numpy as jnp
from jax import lax
from jax.experimental import pallas as pl
from jax.experimental.pallas import tpu as pltpu
from jax.experimental.pallas import tpu_sc as plsc

N = 10000
F = 128
E = 320000
G = 64
NC, NS = 2, 16
NW = NC * NS
EPW = E // NW
K = 80
NCHUNK = 126
NCHUNKP = NCHUNK
NPAD = 10240
TPT = NPAD // NS
R = 2000
NB = N // R

_mesh = plsc.VectorSubcoreMesh(core_axis_name="c", subcore_axis_name="s")


@functools.partial(
    pl.kernel,
    out_type=jax.ShapeDtypeStruct((NC, NPAD), jnp.float32),
    mesh=_mesh,
    scratch_types=[
        pltpu.VMEM((NCHUNKP, K), jnp.int32),
        pltpu.VMEM((K,), jnp.float32),
        pltpu.VMEM_SHARED((NPAD,), jnp.float32),
    ],
)
def _sc_degree(dst_hbm, z_hbm, out_hbm, dst_v, ones_v, acc_sh):
    c = lax.axis_index("c")
    s = lax.axis_index("s")
    w = c * NS + s
    pltpu.sync_copy(z_hbm.at[pl.ds(s * TPT, TPT)], acc_sh.at[pl.ds(s * TPT, TPT)])
    pltpu.sync_copy(dst_hbm.at[w], dst_v)
    for i in range(K // 16):
        ones_v[pl.ds(i * 16, 16)] = jnp.ones((16,), jnp.float32)
    plsc.subcore_barrier()

    def body(j, carry):
        pltpu.sync_copy(ones_v, acc_sh.at[dst_v.at[j]], add=True)
        return carry

    lax.fori_loop(0, NCHUNK, body, 0)
    plsc.subcore_barrier()
    pltpu.sync_copy(acc_sh.at[pl.ds(s * TPT, TPT)],
                    out_hbm.at[c, pl.ds(s * TPT, TPT)])


@functools.partial(
    pl.kernel,
    out_type=jax.ShapeDtypeStruct((NC, NPAD, F), jnp.float32),
    mesh=_mesh,
    scratch_types=[
        pltpu.VMEM((NCHUNKP * K,), jnp.int32),
        pltpu.VMEM((NCHUNKP, K), jnp.int32),
        [pltpu.VMEM((K, F), jnp.float32) for _ in range(2)],
        pltpu.VMEM_SHARED((NPAD, F), jnp.float32),
        [pltpu.SemaphoreType.DMA for _ in range(2)],
    ],
)
def _sc_edge(hs_hbm, src_hbm, dst_hbm, z_hbm, out_hbm,
             src_v, dst_v, rows, acc_sh, gsem):
    c = lax.axis_index("c")
    s = lax.axis_index("s")
    w = c * NS + s
    pltpu.sync_copy(z_hbm.at[pl.ds(s * TPT, TPT)], acc_sh.at[pl.ds(s * TPT, TPT)])
    pltpu.sync_copy(src_hbm.at[w], src_v)
    pltpu.sync_copy(dst_hbm.at[w], dst_v)
    plsc.subcore_barrier()

    def body(i, carry):
        j0 = 2 * i
        o0 = pl.multiple_of(j0 * K, 8 * K)
        o1 = pl.multiple_of(j0 * K + K, K)
        d0 = pltpu.async_copy(hs_hbm.at[src_v.at[pl.ds(o0, K)]],
                              rows[0], gsem[0])
        d1 = pltpu.async_copy(hs_hbm.at[src_v.at[pl.ds(o1, K)]],
                              rows[1], gsem[1])
        d0.wait()
        pltpu.sync_copy(rows[0], acc_sh.at[dst_v.at[j0]], add=True)
        d1.wait()
        pltpu.sync_copy(rows[1], acc_sh.at[dst_v.at[j0 + 1]], add=True)
        return carry

    lax.fori_loop(0, NCHUNK // 2, body, 0)
    plsc.subcore_barrier()
    pltpu.sync_copy(acc_sh.at[pl.ds(s * TPT, TPT)],
                    out_hbm.at[c, pl.ds(s * TPT, TPT)])


def _tc_first(x, W1, cnt_t):
    def body(x_ref, w_ref, c_ref, hs_ref, dis_ref):
        deg = 1.0 + c_ref[:, 0:1] + c_ref[:, 1:2]
        dis = lax.rsqrt(deg)
        h = jnp.dot(x_ref[...], w_ref[...], preferred_element_type=jnp.float32)
        hs_ref[...] = dis * h
        dis_ref[...] = dis

    return pl.pallas_call(
        body,
        grid=(NB,),
        in_specs=[
            pl.BlockSpec((R, F), lambda i: (i, 0)),
            pl.BlockSpec((F, F), lambda i: (0, 0)),
            pl.BlockSpec((R, 2), lambda i: (i, 0)),
        ],
        out_specs=[
            pl.BlockSpec((R, F), lambda i: (i, 0)),
            pl.BlockSpec((R, 1), lambda i: (i, 0)),
        ],
        out_shape=[
            jax.ShapeDtypeStruct((N, F), jnp.float32),
            jax.ShapeDtypeStruct((N, 1), jnp.float32),
        ],
    )(x, W1, cnt_t)


def _tc_mid(p, hs1, dis, b1r, W2):
    def body(p_ref, hs_ref, dis_ref, b_ref, w_ref, out_ref):
        sm = p_ref[0, :, :] + p_ref[1, :, :] + hs_ref[...]
        a = jnp.maximum(dis_ref[...] * sm + b_ref[...], 0.0)
        out_ref[...] = dis_ref[...] * jnp.dot(
            a, w_ref[...], preferred_element_type=jnp.float32)

    return pl.pallas_call(
        body,
        grid=(NB,),
        in_specs=[
            pl.BlockSpec((NC, R, F), lambda i: (0, i, 0)),
            pl.BlockSpec((R, F), lambda i: (i, 0)),
            pl.BlockSpec((R, 1), lambda i: (i, 0)),
            pl.BlockSpec((1, F), lambda i: (0, 0)),
            pl.BlockSpec((F, F), lambda i: (0, 0)),
        ],
        out_specs=pl.BlockSpec((R, F), lambda i: (i, 0)),
        out_shape=jax.ShapeDtypeStruct((N, F), jnp.float32),
    )(p, hs1, dis, b1r, W2)


def _tc_final(q, hs2, dis, b2r, batchf, L1W, l1br, L2W, l2br):
    def body(q_ref, hs_ref, dis_ref, b_ref, bf_ref, w1_ref, bb1_ref,
             w2_ref, bb2_ref, out_ref, G_acc, C_acc):
        i = pl.program_id(0)

        @pl.when(i == 0)
        def _():
            G_acc[...] = jnp.zeros_like(G_acc)
            C_acc[...] = jnp.zeros_like(C_acc)

        sm = q_ref[0, :, :] + q_ref[1, :, :] + hs_ref[...]
        h2 = jnp.maximum(dis_ref[...] * sm + b_ref[...], 0.0)
        gid = lax.broadcasted_iota(jnp.int32, (R, G), 1).astype(jnp.float32)
        oh = (bf_ref[...] == gid).astype(jnp.float32)
        G_acc[...] += lax.dot_general(
            oh, h2, (((0,), (0,)), ((), ())),
            preferred_element_type=jnp.float32)
        C_acc[...] += lax.dot_general(
            oh, jnp.ones((R, 1), jnp.float32), (((0,), (0,)), ((), ())),
            preferred_element_type=jnp.float32)

        @pl.when(i == NB - 1)
        def _():
            g = G_acc[...] / jnp.maximum(C_acc[...], 1.0)
            z = jnp.maximum(
                jnp.dot(g, w1_ref[...], preferred_element_type=jnp.float32)
                + bb1_ref[...], 0.0)
            out_ref[...] = jnp.dot(
                z, w2_ref[...], preferred_element_type=jnp.float32) + bb2_ref[...]

    return pl.pallas_call(
        body,
        grid=(NB,),
        in_specs=[
            pl.BlockSpec((NC, R, F), lambda i: (0, i, 0)),
            pl.BlockSpec((R, F), lambda i: (i, 0)),
            pl.BlockSpec((R, 1), lambda i: (i, 0)),
            pl.BlockSpec((1, F), lambda i: (0, 0)),
            pl.BlockSpec((R, 1), lambda i: (i, 0)),
            pl.BlockSpec((F, 64), lambda i: (0, 0)),
            pl.BlockSpec((1, 64), lambda i: (0, 0)),
            pl.BlockSpec((64, 40), lambda i: (0, 0)),
            pl.BlockSpec((1, 40), lambda i: (0, 0)),
        ],
        out_specs=pl.BlockSpec((G, 40), lambda i: (0, 0)),
        out_shape=jax.ShapeDtypeStruct((G, 40), jnp.float32),
        scratch_shapes=[
            pltpu.VMEM((G, F), jnp.float32),
            pltpu.VMEM((G, 1), jnp.float32),
        ],
    )(q, hs2, dis, b2r, batchf, L1W, l1br, L2W, l2br)


def kernel(x, edge_index, batch, W1, b1, W2, b2, L1W, L1b, L2W, L2b):
    pad = NCHUNKP * K - EPW
    src = jnp.pad(edge_index[0].reshape(NW, EPW),
                  ((0, 0), (0, pad)))

    dst = jnp.pad(edge_index[1].reshape(NW, EPW), ((0, 0), (0, pad)),
                  constant_values=NPAD - 1).reshape(NW, NCHUNKP, K)
    z1 = jnp.zeros((NPAD,), jnp.float32)
    z2 = jnp.zeros((NPAD, F), jnp.float32)
    b1r = b1.reshape(1, F)
    b2r = b2.reshape(1, F)
    l1br = L1b.reshape(1, 64)
    l2br = L2b.reshape(1, 40)
    batchf = batch.astype(jnp.float32).reshape(N, 1)

    cnt = _sc_degree(dst, z1)
    cnt_t = cnt.T
    hs1, dis = _tc_first(x, W1, cnt_t)
    p = _sc_edge(hs1, src, dst, z2)
    hs2 = _tc_mid(p, hs1, dis, b1r, W2)
    q = _sc_edge(hs2, src, dst, z2)
    return _tc_final(q, hs2, dis, b2r, batchf, L1W, l1br, L2W, l2br)


# speedup vs baseline: 1.9976x; 1.1844x over previous
import functools

import jax
import jax.numpy as jnp
from jax import lax
from jax.experimental import pallas as pl
from jax.experimental.pallas import tpu as pltpu
from jax.experimental.pallas import tpu_sc as plsc

N = 10000
F = 128
E = 320000
G = 64
NC, NS = 2, 16
NW = NC * NS
EPW = E // NW
K = 80
NCHUNK = 125
NCHUNKP = NCHUNK
NPAD = 10240
TPT = NPAD // NS
R = 2000
NB = N // R

_mesh = plsc.VectorSubcoreMesh(core_axis_name="c", subcore_axis_name="s")


@functools.partial(
    pl.kernel,
    out_type=jax.ShapeDtypeStruct((NC, NPAD), jnp.float32),
    mesh=_mesh,
    scratch_types=[
        pltpu.VMEM((NCHUNKP, K), jnp.int32),
        pltpu.VMEM((K,), jnp.float32),
        pltpu.VMEM_SHARED((NPAD,), jnp.float32),
    ],
)
def _sc_degree(dst_hbm, z_hbm, out_hbm, dst_v, ones_v, acc_sh):
    c = lax.axis_index("c")
    s = lax.axis_index("s")
    w = c * NS + s
    pltpu.sync_copy(z_hbm.at[pl.ds(s * TPT, TPT)], acc_sh.at[pl.ds(s * TPT, TPT)])
    pltpu.sync_copy(dst_hbm.at[w], dst_v)
    for i in range(K // 16):
        ones_v[pl.ds(i * 16, 16)] = jnp.ones((16,), jnp.float32)
    plsc.subcore_barrier()

    def body(j, carry):
        pltpu.sync_copy(ones_v, acc_sh.at[dst_v.at[j]], add=True)
        return carry

    lax.fori_loop(0, NCHUNK, body, 0)
    plsc.subcore_barrier()
    pltpu.sync_copy(acc_sh.at[pl.ds(s * TPT, TPT)],
                    out_hbm.at[c, pl.ds(s * TPT, TPT)])


@functools.partial(
    pl.kernel,
    out_type=jax.ShapeDtypeStruct((NC, NPAD, F), jnp.float32),
    mesh=_mesh,
    scratch_types=[
        pltpu.VMEM((NCHUNKP, K), jnp.int32),
        pltpu.VMEM((NCHUNKP, K), jnp.int32),
        pltpu.VMEM((K, F), jnp.float32),
        pltpu.VMEM_SHARED((NPAD, F), jnp.float32),
        pltpu.SemaphoreType.DMA,
    ],
)
def _sc_edge(hs_hbm, src_hbm, dst_hbm, z_hbm, out_hbm,
             src_v, dst_v, rows_v, acc_sh, sem):
    c = lax.axis_index("c")
    s = lax.axis_index("s")
    w = c * NS + s
    pltpu.sync_copy(z_hbm.at[pl.ds(s * TPT, TPT)], acc_sh.at[pl.ds(s * TPT, TPT)])
    pltpu.sync_copy(src_hbm.at[w], src_v)
    pltpu.sync_copy(dst_hbm.at[w], dst_v)
    plsc.subcore_barrier()

    def body(j, carry):
        pltpu.async_copy(hs_hbm.at[src_v.at[j]], rows_v, sem).wait()
        pltpu.sync_copy(rows_v, acc_sh.at[dst_v.at[j]], add=True)
        return carry

    lax.fori_loop(0, NCHUNK, body, 0)
    plsc.subcore_barrier()
    pltpu.sync_copy(acc_sh.at[pl.ds(s * TPT, TPT)],
                    out_hbm.at[c, pl.ds(s * TPT, TPT)])


def _tc_first(x, W1, cnt_t):
    def body(x_ref, w_ref, c_ref, hs_ref, dis_ref):
        deg = 1.0 + c_ref[:, 0:1] + c_ref[:, 1:2]
        dis = lax.rsqrt(deg)
        h = jnp.dot(x_ref[...], w_ref[...], preferred_element_type=jnp.float32)
        hs_ref[...] = dis * h
        dis_ref[...] = dis

    return pl.pallas_call(
        body,
        grid=(NB,),
        in_specs=[
            pl.BlockSpec((R, F), lambda i: (i, 0)),
            pl.BlockSpec((F, F), lambda i: (0, 0)),
            pl.BlockSpec((R, 2), lambda i: (i, 0)),
        ],
        out_specs=[
            pl.BlockSpec((R, F), lambda i: (i, 0)),
            pl.BlockSpec((R, 1), lambda i: (i, 0)),
        ],
        out_shape=[
            jax.ShapeDtypeStruct((N, F), jnp.float32),
            jax.ShapeDtypeStruct((N, 1), jnp.float32),
        ],
    )(x, W1, cnt_t)


def _tc_mid(p, hs1, dis, b1r, W2):
    def body(p_ref, hs_ref, dis_ref, b_ref, w_ref, out_ref):
        sm = p_ref[0, :, :] + p_ref[1, :, :] + hs_ref[...]
        a = jnp.maximum(dis_ref[...] * sm + b_ref[...], 0.0)
        out_ref[...] = dis_ref[...] * jnp.dot(
            a, w_ref[...], preferred_element_type=jnp.float32)

    return pl.pallas_call(
        body,
        grid=(NB,),
        in_specs=[
            pl.BlockSpec((NC, R, F), lambda i: (0, i, 0)),
            pl.BlockSpec((R, F), lambda i: (i, 0)),
            pl.BlockSpec((R, 1), lambda i: (i, 0)),
            pl.BlockSpec((1, F), lambda i: (0, 0)),
            pl.BlockSpec((F, F), lambda i: (0, 0)),
        ],
        out_specs=pl.BlockSpec((R, F), lambda i: (i, 0)),
        out_shape=jax.ShapeDtypeStruct((N, F), jnp.float32),
    )(p, hs1, dis, b1r, W2)


def _tc_final(q, hs2, dis, b2r, batchf, L1W, l1br, L2W, l2br):
    def body(q_ref, hs_ref, dis_ref, b_ref, bf_ref, w1_ref, bb1_ref,
             w2_ref, bb2_ref, out_ref, G_acc, C_acc):
        i = pl.program_id(0)

        @pl.when(i == 0)
        def _():
            G_acc[...] = jnp.zeros_like(G_acc)
            C_acc[...] = jnp.zeros_like(C_acc)

        sm = q_ref[0, :, :] + q_ref[1, :, :] + hs_ref[...]
        h2 = jnp.maximum(dis_ref[...] * sm + b_ref[...], 0.0)
        gid = lax.broadcasted_iota(jnp.int32, (R, G), 1).astype(jnp.float32)
        oh = (bf_ref[...] == gid).astype(jnp.float32)
        G_acc[...] += lax.dot_general(
            oh, h2, (((0,), (0,)), ((), ())),
            preferred_element_type=jnp.float32)
        C_acc[...] += lax.dot_general(
            oh, jnp.ones((R, 1), jnp.float32), (((0,), (0,)), ((), ())),
            preferred_element_type=jnp.float32)

        @pl.when(i == NB - 1)
        def _():
            g = G_acc[...] / jnp.maximum(C_acc[...], 1.0)
            z = jnp.maximum(
                jnp.dot(g, w1_ref[...], preferred_element_type=jnp.float32)
                + bb1_ref[...], 0.0)
            out_ref[...] = jnp.dot(
                z, w2_ref[...], preferred_element_type=jnp.float32) + bb2_ref[...]

    return pl.pallas_call(
        body,
        grid=(NB,),
        in_specs=[
            pl.BlockSpec((NC, R, F), lambda i: (0, i, 0)),
            pl.BlockSpec((R, F), lambda i: (i, 0)),
            pl.BlockSpec((R, 1), lambda i: (i, 0)),
            pl.BlockSpec((1, F), lambda i: (0, 0)),
            pl.BlockSpec((R, 1), lambda i: (i, 0)),
            pl.BlockSpec((F, 64), lambda i: (0, 0)),
            pl.BlockSpec((1, 64), lambda i: (0, 0)),
            pl.BlockSpec((64, 40), lambda i: (0, 0)),
            pl.BlockSpec((1, 40), lambda i: (0, 0)),
        ],
        out_specs=pl.BlockSpec((G, 40), lambda i: (0, 0)),
        out_shape=jax.ShapeDtypeStruct((G, 40), jnp.float32),
        scratch_shapes=[
            pltpu.VMEM((G, F), jnp.float32),
            pltpu.VMEM((G, 1), jnp.float32),
        ],
    )(q, hs2, dis, b2r, batchf, L1W, l1br, L2W, l2br)


def kernel(x, edge_index, batch, W1, b1, W2, b2, L1W, L1b, L2W, L2b):
    src = edge_index[0].reshape(NW, NCHUNKP, K)
    dst = edge_index[1].reshape(NW, NCHUNKP, K)
    z1 = jnp.zeros((NPAD,), jnp.float32)
    z2 = jnp.zeros((NPAD, F), jnp.float32)
    b1r = b1.reshape(1, F)
    b2r = b2.reshape(1, F)
    l1br = L1b.reshape(1, 64)
    l2br = L2b.reshape(1, 40)
    batchf = batch.astype(jnp.float32).reshape(N, 1)

    cnt = _sc_degree(dst, z1)
    cnt_t = cnt.T
    hs1, dis = _tc_first(x, W1, cnt_t)
    p = _sc_edge(hs1, src, dst, z2)
    hs2 = _tc_mid(p, hs1, dis, b1r, W2)
    q = _sc_edge(hs2, src, dst, z2)
    return _tc_final(q, hs2, dis, b2r, batchf, L1W, l1br, L2W, l2br)


# speedup vs baseline: 2.0010x; 1.0017x over previous
import functools

import jax
import jax.numpy as jnp
from jax import lax
from jax.experimental import pallas as pl
from jax.experimental.pallas import tpu as pltpu
from jax.experimental.pallas import tpu_sc as plsc

N = 10000
F = 128
E = 320000
G = 64
NC, NS = 2, 16
NW = NC * NS
EPW = E // NW
K = 80
NCHUNK = 125
NCHUNKP = NCHUNK
NPAD = 10240
TPT = NPAD // NS
R = 2000
NB = N // R

_mesh = plsc.VectorSubcoreMesh(core_axis_name="c", subcore_axis_name="s")


@functools.partial(
    pl.kernel,
    out_type=jax.ShapeDtypeStruct((NC, NPAD), jnp.float32),
    mesh=_mesh,
    scratch_types=[
        pltpu.VMEM((NCHUNKP, K), jnp.int32),
        pltpu.VMEM((K,), jnp.float32),
        pltpu.VMEM_SHARED((NPAD,), jnp.float32),
    ],
)
def _sc_degree(dst_hbm, z_hbm, out_hbm, dst_v, ones_v, acc_sh):
    c = lax.axis_index("c")
    s = lax.axis_index("s")
    w = c * NS + s
    pltpu.sync_copy(z_hbm.at[pl.ds(s * TPT, TPT)], acc_sh.at[pl.ds(s * TPT, TPT)])
    pltpu.sync_copy(dst_hbm.at[w], dst_v)
    for i in range(K // 16):
        ones_v[pl.ds(i * 16, 16)] = jnp.ones((16,), jnp.float32)
    plsc.subcore_barrier()

    def body(j, carry):
        pltpu.sync_copy(ones_v, acc_sh.at[dst_v.at[j]], add=True)
        return carry

    lax.fori_loop(0, NCHUNK, body, 0)
    plsc.subcore_barrier()
    pltpu.sync_copy(acc_sh.at[pl.ds(s * TPT, TPT)],
                    out_hbm.at[c, pl.ds(s * TPT, TPT)])


@functools.partial(
    pl.kernel,
    out_type=jax.ShapeDtypeStruct((NC, NPAD, F), jnp.float32),
    mesh=_mesh,
    scratch_types=[
        pltpu.VMEM((NCHUNKP, K), jnp.int32),
        pltpu.VMEM((NCHUNKP, K), jnp.int32),
        pltpu.VMEM((K, F), jnp.float32),
        pltpu.VMEM_SHARED((NPAD, F), jnp.float32),
        pltpu.SemaphoreType.DMA,
    ],
)
def _sc_edge(hs_hbm, src_hbm, dst_hbm, z_hbm, out_hbm,
             src_v, dst_v, rows_v, acc_sh, sem):
    c = lax.axis_index("c")
    s = lax.axis_index("s")
    w = c * NS + s
    pltpu.sync_copy(z_hbm.at[pl.ds(s * TPT, TPT)], acc_sh.at[pl.ds(s * TPT, TPT)])
    pltpu.sync_copy(src_hbm.at[w], src_v)
    pltpu.sync_copy(dst_hbm.at[w], dst_v)
    plsc.subcore_barrier()

    def body(j, carry):
        pltpu.async_copy(hs_hbm.at[src_v.at[j]], rows_v, sem).wait()
        pltpu.sync_copy(rows_v, acc_sh.at[dst_v.at[j]], add=True)
        return carry

    lax.fori_loop(0, NCHUNK, body, 0)
    plsc.subcore_barrier()
    pltpu.sync_copy(acc_sh.at[pl.ds(s * TPT, TPT)],
                    out_hbm.at[c, pl.ds(s * TPT, TPT)])


def _tc_mm1(x, W1):
    def body(x_ref, w_ref, h_ref):
        h_ref[...] = jnp.dot(x_ref[...], w_ref[...],
                             preferred_element_type=jnp.float32)

    return pl.pallas_call(
        body,
        grid=(NB,),
        in_specs=[
            pl.BlockSpec((R, F), lambda i: (i, 0)),
            pl.BlockSpec((F, F), lambda i: (0, 0)),
        ],
        out_specs=pl.BlockSpec((R, F), lambda i: (i, 0)),
        out_shape=jax.ShapeDtypeStruct((N, F), jnp.float32),
    )(x, W1)


def _tc_scale1(h1, cnt_t):
    def body(h_ref, c_ref, hs_ref, dis_ref):
        deg = 1.0 + c_ref[:, 0:1] + c_ref[:, 1:2]
        dis = lax.rsqrt(deg)
        hs_ref[...] = dis * h_ref[...]
        dis_ref[...] = dis

    return pl.pallas_call(
        body,
        grid=(NB,),
        in_specs=[
            pl.BlockSpec((R, F), lambda i: (i, 0)),
            pl.BlockSpec((R, 2), lambda i: (i, 0)),
        ],
        out_specs=[
            pl.BlockSpec((R, F), lambda i: (i, 0)),
            pl.BlockSpec((R, 1), lambda i: (i, 0)),
        ],
        out_shape=[
            jax.ShapeDtypeStruct((N, F), jnp.float32),
            jax.ShapeDtypeStruct((N, 1), jnp.float32),
        ],
    )(h1, cnt_t)


def _tc_mid(p, hs1, dis, b1r, W2):
    def body(p_ref, hs_ref, dis_ref, b_ref, w_ref, out_ref):
        sm = p_ref[0, :, :] + p_ref[1, :, :] + hs_ref[...]
        a = jnp.maximum(dis_ref[...] * sm + b_ref[...], 0.0)
        out_ref[...] = dis_ref[...] * jnp.dot(
            a, w_ref[...], preferred_element_type=jnp.float32)

    return pl.pallas_call(
        body,
        grid=(NB,),
        in_specs=[
            pl.BlockSpec((NC, R, F), lambda i: (0, i, 0)),
            pl.BlockSpec((R, F), lambda i: (i, 0)),
            pl.BlockSpec((R, 1), lambda i: (i, 0)),
            pl.BlockSpec((1, F), lambda i: (0, 0)),
            pl.BlockSpec((F, F), lambda i: (0, 0)),
        ],
        out_specs=pl.BlockSpec((R, F), lambda i: (i, 0)),
        out_shape=jax.ShapeDtypeStruct((N, F), jnp.float32),
    )(p, hs1, dis, b1r, W2)


def _tc_final(q, hs2, dis, b2r, batchf, L1W, l1br, L2W, l2br):
    def body(q_ref, hs_ref, dis_ref, b_ref, bf_ref, w1_ref, bb1_ref,
             w2_ref, bb2_ref, out_ref, G_acc, C_acc):
        i = pl.program_id(0)

        @pl.when(i == 0)
        def _():
            G_acc[...] = jnp.zeros_like(G_acc)
            C_acc[...] = jnp.zeros_like(C_acc)

        sm = q_ref[0, :, :] + q_ref[1, :, :] + hs_ref[...]
        h2 = jnp.maximum(dis_ref[...] * sm + b_ref[...], 0.0)
        gid = lax.broadcasted_iota(jnp.int32, (R, G), 1).astype(jnp.float32)
        oh = (bf_ref[...] == gid).astype(jnp.float32)
        G_acc[...] += lax.dot_general(
            oh, h2, (((0,), (0,)), ((), ())),
            preferred_element_type=jnp.float32)
        C_acc[...] += lax.dot_general(
            oh, jnp.ones((R, 1), jnp.float32), (((0,), (0,)), ((), ())),
            preferred_element_type=jnp.float32)

        @pl.when(i == NB - 1)
        def _():
            g = G_acc[...] / jnp.maximum(C_acc[...], 1.0)
            z = jnp.maximum(
                jnp.dot(g, w1_ref[...], preferred_element_type=jnp.float32)
                + bb1_ref[...], 0.0)
            out_ref[...] = jnp.dot(
                z, w2_ref[...], preferred_element_type=jnp.float32) + bb2_ref[...]

    return pl.pallas_call(
        body,
        grid=(NB,),
        in_specs=[
            pl.BlockSpec((NC, R, F), lambda i: (0, i, 0)),
            pl.BlockSpec((R, F), lambda i: (i, 0)),
            pl.BlockSpec((R, 1), lambda i: (i, 0)),
            pl.BlockSpec((1, F), lambda i: (0, 0)),
            pl.BlockSpec((R, 1), lambda i: (i, 0)),
            pl.BlockSpec((F, 64), lambda i: (0, 0)),
            pl.BlockSpec((1, 64), lambda i: (0, 0)),
            pl.BlockSpec((64, 40), lambda i: (0, 0)),
            pl.BlockSpec((1, 40), lambda i: (0, 0)),
        ],
        out_specs=pl.BlockSpec((G, 40), lambda i: (0, 0)),
        out_shape=jax.ShapeDtypeStruct((G, 40), jnp.float32),
        scratch_shapes=[
            pltpu.VMEM((G, F), jnp.float32),
            pltpu.VMEM((G, 1), jnp.float32),
        ],
    )(q, hs2, dis, b2r, batchf, L1W, l1br, L2W, l2br)


def kernel(x, edge_index, batch, W1, b1, W2, b2, L1W, L1b, L2W, L2b):
    src = edge_index[0].reshape(NW, NCHUNKP, K)
    dst = edge_index[1].reshape(NW, NCHUNKP, K)
    z1 = jnp.zeros((NPAD,), jnp.float32)
    z2 = jnp.zeros((NPAD, F), jnp.float32)
    b1r = b1.reshape(1, F)
    b2r = b2.reshape(1, F)
    l1br = L1b.reshape(1, 64)
    l2br = L2b.reshape(1, 40)
    batchf = batch.astype(jnp.float32).reshape(N, 1)

    cnt = _sc_degree(dst, z1)
    h1 = _tc_mm1(x, W1)
    cnt_t = cnt.T
    hs1, dis = _tc_scale1(h1, cnt_t)
    p = _sc_edge(hs1, src, dst, z2)
    hs2 = _tc_mid(p, hs1, dis, b1r, W2)
    q = _sc_edge(hs2, src, dst, z2)
    return _tc_final(q, hs2, dis, b2r, batchf, L1W, l1br, L2W, l2br)


# speedup vs baseline: 2.1444x; 1.0717x over previous
import functools

import jax
import jax.numpy as jnp
from jax import lax
from jax.experimental import pallas as pl
from jax.experimental.pallas import tpu as pltpu
from jax.experimental.pallas import tpu_sc as plsc

N = 10000
F = 128
E = 320000
G = 64
NC, NS = 2, 16
NW = NC * NS
EPW = E // NW
K = 100
NCHUNK = 100
NCHUNKP = NCHUNK
NPAD = 10240
TPT = NPAD // NS
R = 2000
NB = N // R

_mesh = plsc.VectorSubcoreMesh(core_axis_name="c", subcore_axis_name="s")


@functools.partial(
    pl.kernel,
    out_type=jax.ShapeDtypeStruct((NC, NPAD), jnp.float32),
    mesh=_mesh,
    scratch_types=[
        pltpu.VMEM((NCHUNKP, K), jnp.int32),
        pltpu.VMEM((K,), jnp.float32),
        pltpu.VMEM_SHARED((NPAD,), jnp.float32),
    ],
)
def _sc_degree(dst_hbm, z_hbm, ones_hbm, out_hbm, dst_v, ones_v, acc_sh):
    c = lax.axis_index("c")
    s = lax.axis_index("s")
    w = c * NS + s
    pltpu.sync_copy(z_hbm.at[pl.ds(s * TPT, TPT)], acc_sh.at[pl.ds(s * TPT, TPT)])
    pltpu.sync_copy(dst_hbm.at[w], dst_v)
    pltpu.sync_copy(ones_hbm, ones_v)
    plsc.subcore_barrier()

    def body(j, carry):
        pltpu.sync_copy(ones_v, acc_sh.at[dst_v.at[j]], add=True)
        return carry

    lax.fori_loop(0, NCHUNK, body, 0)
    plsc.subcore_barrier()
    pltpu.sync_copy(acc_sh.at[pl.ds(s * TPT, TPT)],
                    out_hbm.at[c, pl.ds(s * TPT, TPT)])


@functools.partial(
    pl.kernel,
    out_type=jax.ShapeDtypeStruct((NC, NPAD, F), jnp.float32),
    mesh=_mesh,
    scratch_types=[
        pltpu.VMEM((NCHUNKP, K), jnp.int32),
        pltpu.VMEM((NCHUNKP, K), jnp.int32),
        pltpu.VMEM((K, F), jnp.float32),
        pltpu.VMEM_SHARED((NPAD, F), jnp.float32),
        pltpu.SemaphoreType.DMA,
    ],
)
def _sc_edge(hs_hbm, src_hbm, dst_hbm, z_hbm, out_hbm,
             src_v, dst_v, rows_v, acc_sh, sem):
    c = lax.axis_index("c")
    s = lax.axis_index("s")
    w = c * NS + s
    pltpu.sync_copy(z_hbm.at[pl.ds(s * TPT, TPT)], acc_sh.at[pl.ds(s * TPT, TPT)])
    pltpu.sync_copy(src_hbm.at[w], src_v)
    pltpu.sync_copy(dst_hbm.at[w], dst_v)
    plsc.subcore_barrier()

    def body(j, carry):
        pltpu.async_copy(hs_hbm.at[src_v.at[j]], rows_v, sem).wait()
        pltpu.sync_copy(rows_v, acc_sh.at[dst_v.at[j]], add=True)
        return carry

    lax.fori_loop(0, NCHUNK, body, 0)
    plsc.subcore_barrier()
    pltpu.sync_copy(acc_sh.at[pl.ds(s * TPT, TPT)],
                    out_hbm.at[c, pl.ds(s * TPT, TPT)])


def _tc_mm1(x, W1):
    def body(x_ref, w_ref, h_ref):
        h_ref[...] = jnp.dot(x_ref[...], w_ref[...],
                             preferred_element_type=jnp.float32)

    return pl.pallas_call(
        body,
        grid=(NB,),
        in_specs=[
            pl.BlockSpec((R, F), lambda i: (i, 0)),
            pl.BlockSpec((F, F), lambda i: (0, 0)),
        ],
        out_specs=pl.BlockSpec((R, F), lambda i: (i, 0)),
        out_shape=jax.ShapeDtypeStruct((N, F), jnp.float32),
    )(x, W1)


def _tc_scale1(h1, cnt_t):
    def body(h_ref, c_ref, hs_ref, dis_ref):
        deg = 1.0 + c_ref[:, 0:1] + c_ref[:, 1:2]
        dis = lax.rsqrt(deg)
        hs_ref[...] = dis * h_ref[...]
        dis_ref[...] = dis

    return pl.pallas_call(
        body,
        grid=(NB,),
        in_specs=[
            pl.BlockSpec((R, F), lambda i: (i, 0)),
            pl.BlockSpec((R, 2), lambda i: (i, 0)),
        ],
        out_specs=[
            pl.BlockSpec((R, F), lambda i: (i, 0)),
            pl.BlockSpec((R, 1), lambda i: (i, 0)),
        ],
        out_shape=[
            jax.ShapeDtypeStruct((N, F), jnp.float32),
            jax.ShapeDtypeStruct((N, 1), jnp.float32),
        ],
    )(h1, cnt_t)


def _tc_mid(p, hs1, dis, b1r, W2):
    def body(p_ref, hs_ref, dis_ref, b_ref, w_ref, out_ref):
        sm = p_ref[0, :, :] + p_ref[1, :, :] + hs_ref[...]
        a = jnp.maximum(dis_ref[...] * sm + b_ref[...], 0.0)
        out_ref[...] = dis_ref[...] * jnp.dot(
            a, w_ref[...], preferred_element_type=jnp.float32)

    return pl.pallas_call(
        body,
        grid=(NB,),
        in_specs=[
            pl.BlockSpec((NC, R, F), lambda i: (0, i, 0)),
            pl.BlockSpec((R, F), lambda i: (i, 0)),
            pl.BlockSpec((R, 1), lambda i: (i, 0)),
            pl.BlockSpec((1, F), lambda i: (0, 0)),
            pl.BlockSpec((F, F), lambda i: (0, 0)),
        ],
        out_specs=pl.BlockSpec((R, F), lambda i: (i, 0)),
        out_shape=jax.ShapeDtypeStruct((N, F), jnp.float32),
    )(p, hs1, dis, b1r, W2)


def _tc_final(q, hs2, dis, b2r, batchf, L1W, l1br, L2W, l2br):
    def body(q_ref, hs_ref, dis_ref, b_ref, bf_ref, w1_ref, bb1_ref,
             w2_ref, bb2_ref, out_ref, G_acc, C_acc):
        i = pl.program_id(0)

        @pl.when(i == 0)
        def _():
            G_acc[...] = jnp.zeros_like(G_acc)
            C_acc[...] = jnp.zeros_like(C_acc)

        sm = q_ref[0, :, :] + q_ref[1, :, :] + hs_ref[...]
        h2 = jnp.maximum(dis_ref[...] * sm + b_ref[...], 0.0)
        gid = lax.broadcasted_iota(jnp.int32, (R, G), 1).astype(jnp.float32)
        oh = (bf_ref[...] == gid).astype(jnp.float32)
        G_acc[...] += lax.dot_general(
            oh, h2, (((0,), (0,)), ((), ())),
            preferred_element_type=jnp.float32)
        C_acc[...] += lax.dot_general(
            oh, jnp.ones((R, 1), jnp.float32), (((0,), (0,)), ((), ())),
            preferred_element_type=jnp.float32)

        @pl.when(i == NB - 1)
        def _():
            g = G_acc[...] / jnp.maximum(C_acc[...], 1.0)
            z = jnp.maximum(
                jnp.dot(g, w1_ref[...], preferred_element_type=jnp.float32)
                + bb1_ref[...], 0.0)
            out_ref[...] = jnp.dot(
                z, w2_ref[...], preferred_element_type=jnp.float32) + bb2_ref[...]

    return pl.pallas_call(
        body,
        grid=(NB,),
        in_specs=[
            pl.BlockSpec((NC, R, F), lambda i: (0, i, 0)),
            pl.BlockSpec((R, F), lambda i: (i, 0)),
            pl.BlockSpec((R, 1), lambda i: (i, 0)),
            pl.BlockSpec((1, F), lambda i: (0, 0)),
            pl.BlockSpec((R, 1), lambda i: (i, 0)),
            pl.BlockSpec((F, 64), lambda i: (0, 0)),
            pl.BlockSpec((1, 64), lambda i: (0, 0)),
            pl.BlockSpec((64, 40), lambda i: (0, 0)),
            pl.BlockSpec((1, 40), lambda i: (0, 0)),
        ],
        out_specs=pl.BlockSpec((G, 40), lambda i: (0, 0)),
        out_shape=jax.ShapeDtypeStruct((G, 40), jnp.float32),
        scratch_shapes=[
            pltpu.VMEM((G, F), jnp.float32),
            pltpu.VMEM((G, 1), jnp.float32),
        ],
    )(q, hs2, dis, b2r, batchf, L1W, l1br, L2W, l2br)


def kernel(x, edge_index, batch, W1, b1, W2, b2, L1W, L1b, L2W, L2b):
    src = edge_index[0].reshape(NW, NCHUNKP, K)
    dst = edge_index[1].reshape(NW, NCHUNKP, K)
    z1 = jnp.zeros((NPAD,), jnp.float32)
    z2 = jnp.zeros((NPAD, F), jnp.float32)
    b1r = b1.reshape(1, F)
    b2r = b2.reshape(1, F)
    l1br = L1b.reshape(1, 64)
    l2br = L2b.reshape(1, 40)
    batchf = batch.astype(jnp.float32).reshape(N, 1)

    ones = jnp.ones((K,), jnp.float32)
    cnt = _sc_degree(dst, z1, ones)
    h1 = _tc_mm1(x, W1)
    cnt_t = cnt.T
    hs1, dis = _tc_scale1(h1, cnt_t)
    p = _sc_edge(hs1, src, dst, z2)
    hs2 = _tc_mid(p, hs1, dis, b1r, W2)
    q = _sc_edge(hs2, src, dst, z2)
    return _tc_final(q, hs2, dis, b2r, batchf, L1W, l1br, L2W, l2br)
